# baseline (speedup 1.0000x reference)
import functools

import jax
import jax.numpy as jnp
from jax.experimental import pallas as pl


_ROW_BLOCK = 1024


def _min_over_rows(x_ref, y_ref, yt_ref, n_rows):
    n = yt_ref.shape[2]
    blk = min(_ROW_BLOCK, n_rows)
    y_bf = y_ref[0].astype(jnp.bfloat16)
    yx = yt_ref[0, 0:1, :]
    yy = yt_ref[0, 1:2, :]
    yz = yt_ref[0, 2:3, :]
    y2 = yx * yx + yy * yy + yz * yz

    def step(i, acc):
        xb = x_ref[0, pl.ds(i * blk, blk), :]
        xy = jax.lax.dot_general(
            xb.astype(jnp.bfloat16), y_bf, (((1,), (1,)), ((), ())),
            preferred_element_type=jnp.float32,
        )
        x2 = jnp.sum(xb * xb, axis=1, keepdims=True)
        d = (x2 + y2) - 2.0 * xy
        return jnp.minimum(acc, jnp.min(d, axis=0, keepdims=True))

    acc0 = jnp.full((1, n), jnp.inf, dtype=jnp.float32)
    acc = jax.lax.fori_loop(0, n_rows // blk, step, acc0)
    return jnp.sqrt(jnp.maximum(acc, 0.0))


def _topk_sum(x, k):
    bits = jax.lax.bitcast_convert_type(x, jnp.int32)

    def bs(_, lohi):
        lo, hi = lohi
        mid = lo + (hi - lo + 1) // 2
        cnt = jnp.sum((bits >= mid).astype(jnp.int32))
        take = cnt >= k
        return jnp.where(take, mid, lo), jnp.where(take, hi, mid - 1)

    lo, _ = jax.lax.fori_loop(
        0, 32, bs, (jnp.int32(0), jnp.int32(0x7F000000)))
    v = jax.lax.bitcast_convert_type(lo, jnp.float32)
    sum_gt = jnp.sum(jnp.where(x > v, x, 0.0))
    cnt_gt = jnp.sum((x > v).astype(jnp.float32))
    return sum_gt + (jnp.float32(k) - cnt_gt) * v


def _loss_kernel(p_ref, g_ref, pred_t_ref, gt_t_ref, out_ref, *, n, k):
    dist2 = _min_over_rows(p_ref, g_ref, gt_t_ref, n)
    dist1 = _min_over_rows(g_ref, p_ref, pred_t_ref, n)
    inv_n = jnp.float32(1.0 / n)
    loss_cd = (jnp.sum(dist1) + jnp.sum(dist2)) * inv_n
    loss_w = (_topk_sum(dist1, k) + _topk_sum(dist2, k)) * jnp.float32(1.0 / k)
    out_ref[0, 0, :] = jnp.full((128,), loss_cd + 3.0 * loss_w, jnp.float32)


def kernel(pred_pointclouds, gt_pointclouds):
    pred = pred_pointclouds.astype(jnp.float32)
    gt = gt_pointclouds.astype(jnp.float32)
    b, n, _ = pred.shape
    k = int(0.5 * n)

    zpad = jnp.zeros((b, n, 5), jnp.float32)
    p_pad = jnp.concatenate([pred, zpad], axis=-1)
    g_pad = jnp.concatenate([gt, zpad], axis=-1)
    zpad_t = jnp.zeros((b, 5, n), jnp.float32)
    p_t = jnp.concatenate([pred.transpose(0, 2, 1), zpad_t], axis=1)
    g_t = jnp.concatenate([gt.transpose(0, 2, 1), zpad_t], axis=1)

    spec = pl.BlockSpec((1, n, 8), lambda i: (i, 0, 0))
    spec_t = pl.BlockSpec((1, 8, n), lambda i: (i, 0, 0))
    losses = pl.pallas_call(
        functools.partial(_loss_kernel, n=n, k=k),
        grid=(b,),
        in_specs=[spec, spec, spec_t, spec_t],
        out_specs=pl.BlockSpec((1, 1, 128), lambda i: (i, 0, 0)),
        out_shape=jax.ShapeDtypeStruct((b, 1, 128), jnp.float32),
    )(p_pad, g_pad, p_t, g_t)
    return jnp.sum(losses[:, 0, 0]) / b
